# baseline (speedup 1.0000x reference)
import jax
import jax.numpy as jnp
from jax.experimental import pallas as pl
from jax.experimental.pallas import tpu as pltpu

K = 15
NSLOT = 32


def _ce(s, i, j):
    hi = jnp.maximum(s[i], s[j])
    lo = jnp.minimum(s[i], s[j])
    s[i] = hi
    s[j] = lo


def _bitonic_sort_desc(s, lo, n):
    k = 2
    while k <= n:
        j = k // 2
        while j >= 1:
            for ri in range(n):
                rl = ri ^ j
                if rl > ri:
                    if (ri & k) == 0:
                        _ce(s, lo + ri, lo + rl)
                    else:
                        _ce(s, lo + rl, lo + ri)
            j //= 2
        k *= 2


def _top16_sorted(s):
    _bitonic_sort_desc(s, 0, 16)
    _bitonic_sort_desc(s, 16, 16)
    c = [jnp.maximum(s[i], s[31 - i]) for i in range(16)]
    for j in (8, 4, 2, 1):
        for i in range(16):
            l = i ^ j
            if l > i:
                _ce(c, i, l)
    return c


def _threshold(a_slots):
    s = _top16_sorted(a_slots)
    m = jnp.max(s[0], axis=1, keepdims=True)
    for k in range(1, K):
        sel = s[0] == m
        for t in range(K - k):
            s[t] = jnp.where(sel, s[t + 1], s[t])
        m = jnp.max(s[0], axis=1, keepdims=True)
    return m


def _pipelined_body(
    x_ref, w_ref, wt_ref, eb_ref, db_ref, rec_ref, zs_ref, zbuf_ref
):
    i = pl.program_id(0)
    n = pl.num_programs(0)

    @pl.when(i < n - 1)
    def _encode():
        zbuf_ref[i % 2] = (
            jnp.dot(x_ref[...], w_ref[...], preferred_element_type=jnp.float32)
            + eb_ref[...][None, :]
        )

    @pl.when(i > 0)
    def _process():
        zv = zbuf_ref[(i + 1) % 2]
        blk = zv.shape[0]
        for c in range(blk // 8):
            zc = zv[c * 8 : (c + 1) * 8, :]
            a = jnp.abs(zc)
            s = [a[:, t * 128 : (t + 1) * 128] for t in range(NSLOT)]
            m = _threshold(s)
            zs_ref[c * 8 : (c + 1) * 8, :] = jnp.where(a >= m, zc, 0.0)
        rec_ref[...] = (
            jnp.dot(
                zs_ref[...].astype(jnp.bfloat16),
                wt_ref[...],
                preferred_element_type=jnp.float32,
            )
            + db_ref[...][None, :]
        )


@jax.jit
def kernel(x, encoder_weights, encoder_bias, decoder_bias):
    batch, input_dim = x.shape
    latent_dim = encoder_weights.shape[1]
    blk = min(256, batch)
    nblk = batch // blk
    grid = (nblk + 1,)
    wb = encoder_weights.astype(jnp.bfloat16)
    wtb = wb.T
    xb = x.astype(jnp.bfloat16)
    rec, zs = pl.pallas_call(
        _pipelined_body,
        grid=grid,
        in_specs=[
            pl.BlockSpec((blk, input_dim), lambda i: (jnp.minimum(i, nblk - 1), 0)),
            pl.BlockSpec((input_dim, latent_dim), lambda i: (0, 0)),
            pl.BlockSpec((latent_dim, input_dim), lambda i: (0, 0)),
            pl.BlockSpec((latent_dim,), lambda i: (0,)),
            pl.BlockSpec((input_dim,), lambda i: (0,)),
        ],
        out_specs=[
            pl.BlockSpec((blk, input_dim), lambda i: (jnp.maximum(i, 1) - 1, 0)),
            pl.BlockSpec((blk, latent_dim), lambda i: (jnp.maximum(i, 1) - 1, 0)),
        ],
        out_shape=[
            jax.ShapeDtypeStruct((batch, input_dim), jnp.float32),
            jax.ShapeDtypeStruct((batch, latent_dim), jnp.float32),
        ],
        scratch_shapes=[pltpu.VMEM((2, blk, latent_dim), jnp.float32)],
    )(xb, wb, wtb, encoder_bias, decoder_bias)
    return rec, zs


# speedup vs baseline: 1.0871x; 1.0871x over previous
import jax
import jax.numpy as jnp
from jax.experimental import pallas as pl
from jax.experimental.pallas import tpu as pltpu

K = 15
NSLOT = 32


def _ce(s, i, j):
    hi = jnp.maximum(s[i], s[j])
    lo = jnp.minimum(s[i], s[j])
    s[i] = hi
    s[j] = lo


def _bitonic_sort_desc(s, lo, n):
    k = 2
    while k <= n:
        j = k // 2
        while j >= 1:
            for ri in range(n):
                rl = ri ^ j
                if rl > ri:
                    if (ri & k) == 0:
                        _ce(s, lo + ri, lo + rl)
                    else:
                        _ce(s, lo + rl, lo + ri)
            j //= 2
        k *= 2


def _top16_sorted(s):
    _bitonic_sort_desc(s, 0, 16)
    _bitonic_sort_desc(s, 16, 16)
    c = [jnp.maximum(s[i], s[31 - i]) for i in range(16)]
    for j in (8, 4, 2, 1):
        for i in range(16):
            l = i ^ j
            if l > i:
                _ce(c, i, l)
    return c


def _threshold(a_slots):
    s = _top16_sorted(a_slots)
    m = jnp.max(s[0], axis=1, keepdims=True)
    for k in range(1, K):
        sel = s[0] == m
        for t in range(K - k):
            s[t] = jnp.where(sel, s[t + 1], s[t])
        m = jnp.max(s[0], axis=1, keepdims=True)
    return m


def _cleanup8(c, lo=0):
    for j in (4, 2, 1):
        for i in range(8):
            l = i ^ j
            if l > i:
                _ce(c, lo + i, lo + l)


def _top8_sorted(s):
    for g in range(4):
        _bitonic_sort_desc(s, 8 * g, 8)
    a = [jnp.maximum(s[i], s[15 - i]) for i in range(8)]
    b = [jnp.maximum(s[16 + i], s[31 - i]) for i in range(8)]
    _cleanup8(a)
    _cleanup8(b)
    c = [jnp.maximum(a[i], b[7 - i]) for i in range(8)]
    _cleanup8(c)
    return c


def _threshold8(a_slots):
    s = _top8_sorted(a_slots)
    cnt = jnp.zeros_like(s[0])
    m = jnp.max(s[0], axis=1, keepdims=True)
    for k in range(1, K):
        sel = s[0] == m
        cnt = cnt + jnp.where(sel, 1.0, 0.0)
        for t in range(min(K - k, 7)):
            s[t] = jnp.where(sel, s[t + 1], s[t])
        m = jnp.max(s[0], axis=1, keepdims=True)
    return m, cnt


def _pipelined_body(
    x_ref, w_ref, wt_ref, eb_ref, db_ref, rec_ref, zs_ref, zbuf_ref
):
    i = pl.program_id(0)
    n = pl.num_programs(0)

    @pl.when(i < n - 1)
    def _encode():
        zbuf_ref[i % 2] = (
            jnp.dot(x_ref[...], w_ref[...], preferred_element_type=jnp.float32)
            + eb_ref[...][None, :]
        )

    @pl.when(i > 0)
    def _process():
        zv = zbuf_ref[(i + 1) % 2]
        blk = zv.shape[0]
        bad = None
        for c in range(blk // 8):
            zc = zv[c * 8 : (c + 1) * 8, :]
            a = jnp.abs(zc)
            s = [a[:, t * 128 : (t + 1) * 128] for t in range(NSLOT)]
            m, cnt = _threshold8(s)
            b = cnt >= 8.0
            bad = b if bad is None else jnp.logical_or(bad, b)
            zs_ref[c * 8 : (c + 1) * 8, :] = jnp.where(a >= m, zc, 0.0)

        @pl.when(jnp.any(bad))
        def _exact_fallback():
            for c in range(blk // 8):
                zc = zv[c * 8 : (c + 1) * 8, :]
                a = jnp.abs(zc)
                s = [a[:, t * 128 : (t + 1) * 128] for t in range(NSLOT)]
                m = _threshold(s)
                zs_ref[c * 8 : (c + 1) * 8, :] = jnp.where(a >= m, zc, 0.0)

        rec_ref[...] = (
            jnp.dot(
                zs_ref[...].astype(jnp.bfloat16),
                wt_ref[...],
                preferred_element_type=jnp.float32,
            )
            + db_ref[...][None, :]
        )


@jax.jit
def kernel(x, encoder_weights, encoder_bias, decoder_bias):
    batch, input_dim = x.shape
    latent_dim = encoder_weights.shape[1]
    blk = min(256, batch)
    nblk = batch // blk
    grid = (nblk + 1,)
    wb = encoder_weights.astype(jnp.bfloat16)
    wtb = wb.T
    xb = x.astype(jnp.bfloat16)
    rec, zs = pl.pallas_call(
        _pipelined_body,
        grid=grid,
        in_specs=[
            pl.BlockSpec((blk, input_dim), lambda i: (jnp.minimum(i, nblk - 1), 0)),
            pl.BlockSpec((input_dim, latent_dim), lambda i: (0, 0)),
            pl.BlockSpec((latent_dim, input_dim), lambda i: (0, 0)),
            pl.BlockSpec((latent_dim,), lambda i: (0,)),
            pl.BlockSpec((input_dim,), lambda i: (0,)),
        ],
        out_specs=[
            pl.BlockSpec((blk, input_dim), lambda i: (jnp.maximum(i, 1) - 1, 0)),
            pl.BlockSpec((blk, latent_dim), lambda i: (jnp.maximum(i, 1) - 1, 0)),
        ],
        out_shape=[
            jax.ShapeDtypeStruct((batch, input_dim), jnp.float32),
            jax.ShapeDtypeStruct((batch, latent_dim), jnp.float32),
        ],
        scratch_shapes=[pltpu.VMEM((2, blk, latent_dim), jnp.float32)],
    )(xb, wb, wtb, encoder_bias, decoder_bias)
    return rec, zs


# speedup vs baseline: 1.1161x; 1.0267x over previous
import jax
import jax.numpy as jnp
from jax.experimental import pallas as pl
from jax.experimental.pallas import tpu as pltpu

K = 15
NSLOT = 32


def _ce(s, i, j):
    hi = jnp.maximum(s[i], s[j])
    lo = jnp.minimum(s[i], s[j])
    s[i] = hi
    s[j] = lo


def _bitonic_sort_desc(s, lo, n):
    k = 2
    while k <= n:
        j = k // 2
        while j >= 1:
            for ri in range(n):
                rl = ri ^ j
                if rl > ri:
                    if (ri & k) == 0:
                        _ce(s, lo + ri, lo + rl)
                    else:
                        _ce(s, lo + rl, lo + ri)
            j //= 2
        k *= 2


def _top16_sorted(s):
    _bitonic_sort_desc(s, 0, 16)
    _bitonic_sort_desc(s, 16, 16)
    c = [jnp.maximum(s[i], s[31 - i]) for i in range(16)]
    for j in (8, 4, 2, 1):
        for i in range(16):
            l = i ^ j
            if l > i:
                _ce(c, i, l)
    return c


def _threshold(a_slots):
    s = _top16_sorted(a_slots)
    m = jnp.max(s[0], axis=1, keepdims=True)
    for k in range(1, K):
        sel = s[0] == m
        for t in range(K - k):
            s[t] = jnp.where(sel, s[t + 1], s[t])
        m = jnp.max(s[0], axis=1, keepdims=True)
    return m


def _cleanup8(c, lo=0):
    for j in (4, 2, 1):
        for i in range(8):
            l = i ^ j
            if l > i:
                _ce(c, lo + i, lo + l)


def _top8_sorted(s):
    for g in range(4):
        _bitonic_sort_desc(s, 8 * g, 8)
    a = [jnp.maximum(s[i], s[15 - i]) for i in range(8)]
    b = [jnp.maximum(s[16 + i], s[31 - i]) for i in range(8)]
    _cleanup8(a)
    _cleanup8(b)
    c = [jnp.maximum(a[i], b[7 - i]) for i in range(8)]
    _cleanup8(c)
    return c


def _cleanup4(c):
    for j in (2, 1):
        for i in range(4):
            l = i ^ j
            if l > i:
                _ce(c, i, l)


def _merge4(a, b):
    c = [jnp.maximum(a[i], b[3 - i]) for i in range(4)]
    _cleanup4(c)
    return c


def _threshold4(a_slots):
    s = a_slots
    for g in range(8):
        _bitonic_sort_desc(s, 4 * g, 4)
    l1 = [
        _merge4(s[8 * g : 8 * g + 4], s[8 * g + 4 : 8 * g + 8])
        for g in range(4)
    ]
    l2 = [_merge4(l1[0], l1[1]), _merge4(l1[2], l1[3])]
    c = _merge4(l2[0], l2[1])
    cnt = jnp.zeros_like(c[0])
    m = jnp.max(c[0], axis=1, keepdims=True)
    for k in range(1, K):
        sel = c[0] == m
        cnt = cnt + jnp.where(sel, 1.0, 0.0)
        for t in range(min(K - k, 3)):
            c[t] = jnp.where(sel, c[t + 1], c[t])
        m = jnp.max(c[0], axis=1, keepdims=True)
    return m, cnt


def _threshold8(a_slots):
    s = _top8_sorted(a_slots)
    cnt = jnp.zeros_like(s[0])
    m = jnp.max(s[0], axis=1, keepdims=True)
    for k in range(1, K):
        sel = s[0] == m
        cnt = cnt + jnp.where(sel, 1.0, 0.0)
        for t in range(min(K - k, 7)):
            s[t] = jnp.where(sel, s[t + 1], s[t])
        m = jnp.max(s[0], axis=1, keepdims=True)
    return m, cnt


def _pipelined_body(
    x_ref, w_ref, wt_ref, eb_ref, db_ref, rec_ref, zs_ref, zbuf_ref
):
    i = pl.program_id(0)
    n = pl.num_programs(0)

    @pl.when(i < n - 1)
    def _encode():
        zbuf_ref[i % 2] = (
            jnp.dot(x_ref[...], w_ref[...], preferred_element_type=jnp.float32)
            + eb_ref[...][None, :]
        )

    @pl.when(i > 0)
    def _process():
        zv = zbuf_ref[(i + 1) % 2]
        blk = zv.shape[0]
        bad = None
        for c in range(blk // 8):
            zc = zv[c * 8 : (c + 1) * 8, :]
            a = jnp.abs(zc)
            s = [a[:, t * 128 : (t + 1) * 128] for t in range(NSLOT)]
            m, cnt = _threshold4(s)
            b = cnt >= 4.0
            bad = b if bad is None else jnp.logical_or(bad, b)
            zs_ref[c * 8 : (c + 1) * 8, :] = jnp.where(a >= m, zc, 0.0)

        @pl.when(jnp.any(bad))
        def _exact_fallback():
            for c in range(blk // 8):
                zc = zv[c * 8 : (c + 1) * 8, :]
                a = jnp.abs(zc)
                s = [a[:, t * 128 : (t + 1) * 128] for t in range(NSLOT)]
                m = _threshold(s)
                zs_ref[c * 8 : (c + 1) * 8, :] = jnp.where(a >= m, zc, 0.0)

        rec_ref[...] = (
            jnp.dot(
                zs_ref[...].astype(jnp.bfloat16),
                wt_ref[...],
                preferred_element_type=jnp.float32,
            )
            + db_ref[...][None, :]
        )


@jax.jit
def kernel(x, encoder_weights, encoder_bias, decoder_bias):
    batch, input_dim = x.shape
    latent_dim = encoder_weights.shape[1]
    blk = min(256, batch)
    nblk = batch // blk
    grid = (nblk + 1,)
    wb = encoder_weights.astype(jnp.bfloat16)
    wtb = wb.T
    xb = x.astype(jnp.bfloat16)
    rec, zs = pl.pallas_call(
        _pipelined_body,
        grid=grid,
        in_specs=[
            pl.BlockSpec((blk, input_dim), lambda i: (jnp.minimum(i, nblk - 1), 0)),
            pl.BlockSpec((input_dim, latent_dim), lambda i: (0, 0)),
            pl.BlockSpec((latent_dim, input_dim), lambda i: (0, 0)),
            pl.BlockSpec((latent_dim,), lambda i: (0,)),
            pl.BlockSpec((input_dim,), lambda i: (0,)),
        ],
        out_specs=[
            pl.BlockSpec((blk, input_dim), lambda i: (jnp.maximum(i, 1) - 1, 0)),
            pl.BlockSpec((blk, latent_dim), lambda i: (jnp.maximum(i, 1) - 1, 0)),
        ],
        out_shape=[
            jax.ShapeDtypeStruct((batch, input_dim), jnp.float32),
            jax.ShapeDtypeStruct((batch, latent_dim), jnp.float32),
        ],
        scratch_shapes=[pltpu.VMEM((2, blk, latent_dim), jnp.float32)],
    )(xb, wb, wtb, encoder_bias, decoder_bias)
    return rec, zs
